# baseline (speedup 1.0000x reference)
import functools

import jax
import jax.numpy as jnp
from jax import lax
from jax.experimental import pallas as pl
from jax.experimental.pallas import tpu as pltpu
from jax.experimental.pallas import tpu_sc as plsc

T = 2048
D = 2048
L = 32768
K = 64

NC = 2
NS = 16
NW = NC * NS
NCHUNK = 4
CT = T // NCHUNK
TPW = CT // NW
NBLK = 256
NBM = NBLK // 16
CPC = L // NBLK
CAND_CAP = 1024
NEG = -3.0e38
NROWS = 8
NBATCH = K // NROWS
DCH = D // 16

_BL_ENC = 512
_CTM = 128


def _enc_body(x_ref, w_ref, b_ref, o_ref):
    acc = jax.lax.dot_general(
        x_ref[...], w_ref[...], (((1,), (1,)), ((), ())),
        preferred_element_type=jnp.float32)
    o_ref[...] = jnp.maximum(acc + b_ref[...], 0.0)


def _clsmax_body(p_ref, o_ref):
    p = p_ref[...]
    o_ref[...] = jnp.max(p.reshape(_CTM, L // NBLK, NBLK), axis=1)


def _xstats_body(x_ref, o_ref):
    j = pl.program_id(0)

    @pl.when(j == 0)
    def _init():
        o_ref[0, 0] = 0.0

    xb = x_ref[...]
    s = jnp.sum(xb, axis=0)
    sq = jnp.sum(xb * xb, axis=0)
    o_ref[0, 0] += jnp.sum(sq - s * s * (1.0 / T))


def _merge2(ka, va, kb, vb):
    kbr = lax.rev(kb, (0,))
    vbr = lax.rev(vb, (0,))
    m = ka >= kbr
    khi = jnp.where(m, ka, kbr)
    vhi = jnp.where(m, va, vbr)
    klo = jnp.where(m, kbr, ka)
    vlo = jnp.where(m, vbr, va)
    khi, vhi = plsc.sort_key_val(khi, vhi, descending=True)
    klo, vlo = plsc.sort_key_val(klo, vlo, descending=True)
    return khi, vhi, klo, vlo


def _cascade_insert(R, kv, vv):
    ck, cv = plsc.sort_key_val(kv, vv, descending=True)
    out = []
    for i in range(4):
        khi, vhi, ck, cv = _merge2(R[2 * i], R[2 * i + 1], ck, cv)
        out.append(khi)
        out.append(vhi)
    return tuple(out)


def _maybe_insert(R, kv, vv, gate_splat):
    pc = plsc.all_reduce_population_count(kv >= gate_splat)
    return lax.cond(pc[0] > 0,
                    lambda: _cascade_insert(R, kv, vv),
                    lambda: R)


def _sc_body(pre_hbm, bm_hbm, x_hbm, wdec_hbm, bdec_hbm,
             acts_hbm, idx_hbm, out_hbm, esq_hbm,
             databuf2, bmwork, blkbuf, candv, candi, stage_a, stage_i,
             wrows0, wrows1, acc0, acc1, xbuf, bdecbuf, esqbuf,
             semd0, semd1, sem0, sem1, semx, semo0, semo1):
    cid = lax.axis_index("c")
    sid = lax.axis_index("s")
    wid = sid * NC + cid
    t0 = wid * TPW
    iota = lax.iota(jnp.int32, 16)
    neg_splat = jnp.full((16,), NEG, jnp.float32)
    zero_i = jnp.zeros((16,), jnp.int32)

    pltpu.sync_copy(bdec_hbm, bdecbuf)
    pltpu.sync_copy(bm_hbm.at[pl.ds(t0, TPW)], bmwork)
    pltpu.async_copy(pre_hbm.at[t0], databuf2.at[0], semd0)

    def token_step(tt, esq):
        t = t0 + tt
        par = tt % 2
        cpx = pltpu.async_copy(x_hbm.at[t], xbuf, semx)

        @pl.when(jnp.logical_and(tt + 1 < TPW, par == 0))
        def _pf1():
            pltpu.async_copy(pre_hbm.at[t + 1], databuf2.at[1], semd1)

        @pl.when(jnp.logical_and(tt + 1 < TPW, par == 1))
        def _pf0():
            pltpu.async_copy(pre_hbm.at[t + 1], databuf2.at[0], semd0)

        bmv = [bmwork[tt, pl.ds(i * 16, 16)] for i in range(NBM)]
        R = (neg_splat, zero_i) * 4
        for i in range(NBM // 2):
            R = _cascade_insert(R, jnp.maximum(bmv[i], bmv[i + 8]), zero_i)
        tau0 = R[6][15]
        tau0_splat = jnp.full((16,), tau0, jnp.float32)

        cnt_c = jnp.int32(0)
        for i in range(NBM):
            m = bmv[i] >= tau0_splat
            plsc.store_compressed(blkbuf.at[pl.ds(cnt_c, 16)], iota + i * 16,
                                  mask=m)
            pc = plsc.all_reduce_population_count(m)
            cnt_c = cnt_c + pc[0]
        nblk = cnt_c

        @pl.when(par == 0)
        def _wd0():
            pltpu.make_async_copy(pre_hbm.at[t], databuf2.at[0], semd0).wait()

        @pl.when(par == 1)
        def _wd1():
            pltpu.make_async_copy(pre_hbm.at[t], databuf2.at[1], semd1).wait()

        par_splat = jnp.full((16,), par, jnp.int32)

        def flush(R, cnt):
            nv = (cnt + 15) // 16
            def fl_body(q, R):
                base = q * 16
                kv = candv[pl.ds(base, 16)]
                vv = candi[pl.ds(base, 16)]
                valid = (iota + base) < cnt
                kv = jnp.where(valid, kv, neg_splat)
                gate = jnp.full((16,), R[6][15], jnp.float32)
                return _maybe_insert(R, kv, vv, gate)
            return lax.fori_loop(0, nv, fl_body, R)

        R = (neg_splat, zero_i) * 4
        nq = (nblk + 15) // 16

        def ph_d(q, carry):
            R = carry[:8]
            cnt = carry[8]
            clsvec = blkbuf[pl.ds(q * 16, 16)]
            valid = (q * 16 + iota) < nblk

            def inner(s, carry2):
                R, cnt = carry2[:8], carry2[8]
                for u in range(8):
                    idxv = clsvec + (s * 8 + u) * NBLK
                    v = plsc.load_gather(databuf2, [par_splat, idxv],
                                         mask=valid)
                    m = jnp.logical_and(valid, v >= tau0_splat)
                    plsc.store_compressed(candv.at[pl.ds(cnt, 16)], v,
                                          mask=m)
                    plsc.store_compressed(candi.at[pl.ds(cnt, 16)], idxv,
                                          mask=m)
                    pc = plsc.all_reduce_population_count(m)
                    cnt = cnt + pc[0]
                R, cnt = lax.cond(
                    cnt >= CAND_CAP - 128,
                    lambda: (flush(R, cnt), jnp.int32(0)),
                    lambda: (R, cnt))
                return (*R, cnt)

            return lax.fori_loop(0, CPC // 8, inner, (*R, cnt))

        carry = lax.fori_loop(0, nq, ph_d, (*R, jnp.int32(0)))
        R = flush(carry[:8], carry[8])

        for i in range(4):
            stage_a[tt, pl.ds(i * 16, 16)] = R[2 * i]
            stage_i[tt, pl.ds(i * 16, 16)] = R[2 * i + 1]

        act_vregs = [R[0], R[2], R[4], R[6]]
        wbufs = [wrows0, wrows1]
        sems = [sem0, sem1]

        def batch_idx(b):
            return stage_i.at[tt, pl.ds(b * NROWS, NROWS)]

        def decode_with(acc_ref, osem, esq=esq):
            @pl.when(tt >= 2)
            def _drain():
                pltpu.make_async_copy(acc_ref, out_hbm.at[t - 2], osem).wait()

            cp = pltpu.async_copy(wdec_hbm.at[batch_idx(0)], wbufs[0], sem0)
            for b in range(NBATCH):
                if b + 1 < NBATCH:
                    cp_next = pltpu.async_copy(
                        wdec_hbm.at[batch_idx(b + 1)], wbufs[(b + 1) % 2],
                        sems[(b + 1) % 2])
                cp.wait()
                wbuf = wbufs[b % 2]
                a = act_vregs[b // 2]
                lane0 = (b % 2) * NROWS
                splats = [jnp.full((16,), a[lane0 + r], jnp.float32)
                          for r in range(NROWS)]

                def fma_chunk(c, _, wbuf=wbuf, splats=splats, first=(b == 0)):
                    off = c * 16
                    terms = [splats[r] * wbuf[r, pl.ds(off, 16)]
                             for r in range(NROWS)]
                    if first:
                        terms.append(bdecbuf[pl.ds(off, 16)])
                    while len(terms) > 1:
                        terms = [terms[i] + terms[i + 1]
                                 for i in range(0, len(terms) - 1, 2)] + \
                                ([terms[-1]] if len(terms) % 2 else [])
                    if first:
                        acc_ref[pl.ds(off, 16)] = terms[0]
                    else:
                        acc_ref[pl.ds(off, 16)] += terms[0]
                    return 0
                lax.fori_loop(0, DCH, fma_chunk, 0, unroll=4)
                if b + 1 < NBATCH:
                    cp = cp_next

            cpx.wait()

            def esq_chunk(c, esq):
                off = c * 16
                e = acc_ref[pl.ds(off, 16)] - xbuf[pl.ds(off, 16)]
                return esq + e * e
            esq = lax.fori_loop(0, DCH, esq_chunk, esq, unroll=4)
            pltpu.async_copy(acc_ref, out_hbm.at[t], osem)
            return esq

        return lax.cond(
            par == 0,
            lambda: decode_with(acc0, semo0),
            lambda: decode_with(acc1, semo1))

    esq = lax.fori_loop(0, TPW, token_step, jnp.zeros((16,), jnp.float32))
    pltpu.make_async_copy(acc0, out_hbm.at[t0 + TPW - 2], semo0).wait()
    pltpu.make_async_copy(acc1, out_hbm.at[t0 + TPW - 1], semo1).wait()
    esqbuf[...] = esq
    pltpu.sync_copy(stage_a, acts_hbm.at[pl.ds(t0, TPW)])
    pltpu.sync_copy(stage_i, idx_hbm.at[pl.ds(t0, TPW)])
    pltpu.sync_copy(esqbuf, esq_hbm.at[wid])


def _sc_topk_decode(pre_acts, bm, x, W_dec, b_dec):
    mesh = plsc.VectorSubcoreMesh(
        core_axis_name="c", subcore_axis_name="s",
        num_cores=NC, num_subcores=NS)
    fn = pl.kernel(
        _sc_body,
        out_type=(
            jax.ShapeDtypeStruct((CT, K), jnp.float32),
            jax.ShapeDtypeStruct((CT, K), jnp.int32),
            jax.ShapeDtypeStruct((CT, D), jnp.float32),
            jax.ShapeDtypeStruct((NW, 16), jnp.float32),
        ),
        mesh=mesh,
        compiler_params=pltpu.CompilerParams(needs_layout_passes=False),
        scratch_types=[
            pltpu.VMEM((2, L), jnp.float32),
            pltpu.VMEM((TPW, NBLK), jnp.float32),
            pltpu.VMEM((NBLK + 16,), jnp.int32),
            pltpu.VMEM((CAND_CAP + 128,), jnp.float32),
            pltpu.VMEM((CAND_CAP + 128,), jnp.int32),
            pltpu.VMEM((TPW, K), jnp.float32),
            pltpu.VMEM((TPW, K), jnp.int32),
            pltpu.VMEM((NROWS, D), jnp.float32),
            pltpu.VMEM((NROWS, D), jnp.float32),
            pltpu.VMEM((D,), jnp.float32),
            pltpu.VMEM((D,), jnp.float32),
            pltpu.VMEM((D,), jnp.float32),
            pltpu.VMEM((D,), jnp.float32),
            pltpu.VMEM((16,), jnp.float32),
            pltpu.SemaphoreType.DMA,
            pltpu.SemaphoreType.DMA,
            pltpu.SemaphoreType.DMA,
            pltpu.SemaphoreType.DMA,
            pltpu.SemaphoreType.DMA,
            pltpu.SemaphoreType.DMA,
            pltpu.SemaphoreType.DMA,
        ],
    )
    return fn(pre_acts, bm, x, W_dec, b_dec)


def kernel(x, W_enc, b_enc, W_dec, b_dec):
    sae_in = x - b_dec[None, :]

    enc = pl.pallas_call(
        _enc_body,
        grid=(L // _BL_ENC,),
        in_specs=[
            pl.BlockSpec((CT, D), lambda j: (0, 0)),
            pl.BlockSpec((_BL_ENC, D), lambda j: (j, 0)),
            pl.BlockSpec((1, _BL_ENC), lambda j: (0, j)),
        ],
        out_specs=pl.BlockSpec((CT, _BL_ENC), lambda j: (0, j)),
        out_shape=jax.ShapeDtypeStruct((CT, L), jnp.float32),
    )
    clsmax = pl.pallas_call(
        _clsmax_body,
        grid=(CT // _CTM,),
        in_specs=[pl.BlockSpec((_CTM, L), lambda j: (j, 0))],
        out_specs=pl.BlockSpec((_CTM, NBLK), lambda j: (j, 0)),
        out_shape=jax.ShapeDtypeStruct((CT, NBLK), jnp.float32),
    )
    b_enc2 = b_enc.reshape(1, L)

    acts_c, idx_c, out_c, esq_c = [], [], [], []
    for c in range(NCHUNK):
        xc = lax.slice(x, (c * CT, 0), ((c + 1) * CT, D))
        pre_c = enc(lax.slice(sae_in, (c * CT, 0), ((c + 1) * CT, D)),
                    W_enc, b_enc2)
        bm_c = clsmax(pre_c)
        a, i, o, e = _sc_topk_decode(pre_c, bm_c, xc, W_dec, b_dec)
        acts_c.append(a)
        idx_c.append(i)
        out_c.append(o)
        esq_c.append(e)

    top_acts = jnp.concatenate(acts_c, axis=0)
    top_indices = jnp.concatenate(idx_c, axis=0)
    sae_out = jnp.concatenate(out_c, axis=0)

    total_variance = pl.pallas_call(
        _xstats_body,
        grid=(4,),
        in_specs=[pl.BlockSpec((T, D // 4), lambda j: (0, j))],
        out_specs=pl.BlockSpec(memory_space=pltpu.SMEM),
        out_shape=jax.ShapeDtypeStruct((1, 1), jnp.float32),
    )(x)[0, 0]

    fvu = sum(jnp.sum(e) for e in esq_c) / total_variance
    auxk_loss = jnp.zeros(())
    multi_topk_fvu = jnp.zeros(())
    return sae_out, top_acts, top_indices, fvu, auxk_loss, multi_topk_fvu


# speedup vs baseline: 1.0612x; 1.0612x over previous
import functools

import jax
import jax.numpy as jnp
from jax import lax
from jax.experimental import pallas as pl
from jax.experimental.pallas import tpu as pltpu
from jax.experimental.pallas import tpu_sc as plsc

T = 2048
D = 2048
L = 32768
K = 64

NC = 2
NS = 16
NW = NC * NS
NCHUNK = 4
CT = T // NCHUNK
TPW = CT // NW
NSUB = L // 16
NGRP = NSUB // 16
CAND_CAP = 1024
NEG = -3.0e38
NROWS = 8
NBATCH = K // NROWS
DCH = D // 16

_BL_ENC = 512
_CTM = 128


def _enc_body(x_ref, w_ref, b_ref, o_ref):
    acc = jax.lax.dot_general(
        x_ref[...], w_ref[...], (((1,), (1,)), ((), ())),
        preferred_element_type=jnp.float32)
    o_ref[...] = jnp.maximum(acc + b_ref[...], 0.0)


def _xstats_body(x_ref, o_ref):
    j = pl.program_id(0)

    @pl.when(j == 0)
    def _init():
        o_ref[0, 0] = 0.0

    xb = x_ref[...]
    s = jnp.sum(xb, axis=0)
    sq = jnp.sum(xb * xb, axis=0)
    o_ref[0, 0] += jnp.sum(sq - s * s * (1.0 / T))


def _merge2(ka, va, kb, vb):
    kbr = lax.rev(kb, (0,))
    vbr = lax.rev(vb, (0,))
    m = ka >= kbr
    khi = jnp.where(m, ka, kbr)
    vhi = jnp.where(m, va, vbr)
    klo = jnp.where(m, kbr, ka)
    vlo = jnp.where(m, vbr, va)
    khi, vhi = plsc.sort_key_val(khi, vhi, descending=True)
    klo, vlo = plsc.sort_key_val(klo, vlo, descending=True)
    return khi, vhi, klo, vlo


def _cascade_insert(R, kv, vv):
    ck, cv = plsc.sort_key_val(kv, vv, descending=True)
    out = []
    for i in range(4):
        khi, vhi, ck, cv = _merge2(R[2 * i], R[2 * i + 1], ck, cv)
        out.append(khi)
        out.append(vhi)
    return tuple(out)


def _maybe_insert(R, kv, vv, gate_splat):
    pc = plsc.all_reduce_population_count(kv >= gate_splat)
    return lax.cond(pc[0] > 0,
                    lambda: _cascade_insert(R, kv, vv),
                    lambda: R)


def _sc_body(pre_hbm, x_hbm, wdec_hbm, bdec_hbm,
             acts_hbm, idx_hbm, out_hbm, esq_hbm,
             databuf2, bmbuf, subbuf, candv, candi, stage_a, stage_i,
             wrows0, wrows1, acc0, acc1, xbuf, bdecbuf, esqbuf,
             semd0, semd1, sem0, sem1, semx, semo0, semo1):
    cid = lax.axis_index("c")
    sid = lax.axis_index("s")
    wid = sid * NC + cid
    t0 = wid * TPW
    iota = lax.iota(jnp.int32, 16)
    neg_splat = jnp.full((16,), NEG, jnp.float32)
    zero_i = jnp.zeros((16,), jnp.int32)

    pltpu.sync_copy(bdec_hbm, bdecbuf)
    pltpu.async_copy(pre_hbm.at[t0], databuf2.at[0], semd0)

    def token_step(tt, esq):
        t = t0 + tt
        par = tt % 2
        cpx = pltpu.async_copy(x_hbm.at[t], xbuf, semx)

        @pl.when(jnp.logical_and(tt + 1 < TPW, par == 0))
        def _pf1():
            pltpu.async_copy(pre_hbm.at[t + 1], databuf2.at[1], semd1)

        @pl.when(jnp.logical_and(tt + 1 < TPW, par == 1))
        def _pf0():
            pltpu.async_copy(pre_hbm.at[t + 1], databuf2.at[0], semd0)

        @pl.when(par == 0)
        def _wd0():
            pltpu.make_async_copy(pre_hbm.at[t], databuf2.at[0], semd0).wait()

        @pl.when(par == 1)
        def _wd1():
            pltpu.make_async_copy(pre_hbm.at[t], databuf2.at[1], semd1).wait()

        par_splat = jnp.full((16,), par, jnp.int32)

        def ph_a(g, _):
            base = g * 256
            m = databuf2[par, pl.ds(base, 16)]
            for i in range(1, 16):
                m = jnp.maximum(m, databuf2[par, pl.ds(base + i * 16, 16)])
            bmbuf[pl.ds(g * 16, 16)] = m
            return 0
        lax.fori_loop(0, NGRP, ph_a, 0, unroll=2)

        bm2 = []
        for i in range(8):
            m = bmbuf[pl.ds(i * 256, 16)]
            for j in range(1, 16):
                m = jnp.maximum(m, bmbuf[pl.ds(i * 256 + j * 16, 16)])
            bm2.append(m)

        R = (neg_splat, zero_i) * 4
        for i in range(8):
            R = _cascade_insert(R, bm2[i], zero_i)
        tau0 = R[6][15]
        tau0_splat = jnp.full((16,), tau0, jnp.float32)

        def ph_c(g, cnt):
            v = bmbuf[pl.ds(g * 16, 16)]
            m = v >= tau0_splat
            plsc.store_compressed(subbuf.at[pl.ds(cnt, 16)], iota + g * 16,
                                  mask=m)
            pc = plsc.all_reduce_population_count(m)
            return cnt + pc[0]
        nsub = lax.fori_loop(0, NGRP, ph_c, jnp.int32(0))

        def flush(R, cnt):
            nv = (cnt + 15) // 16
            def fl_body(q, R):
                base = q * 16
                kv = candv[pl.ds(base, 16)]
                vv = candi[pl.ds(base, 16)]
                valid = (iota + base) < cnt
                kv = jnp.where(valid, kv, neg_splat)
                gate = jnp.full((16,), R[6][15], jnp.float32)
                return _maybe_insert(R, kv, vv, gate)
            return lax.fori_loop(0, nv, fl_body, R)

        R = (neg_splat, zero_i) * 4

        def ph_d(j, carry):
            R = carry[:8]
            cnt = carry[8]
            sub = subbuf[pl.ds(j, 16)][0]
            base = (sub >> 4) * 256 + (sub & 15)
            idxv = base + iota * 16
            v = plsc.load_gather(databuf2, [par_splat, idxv])
            m = v >= tau0_splat
            plsc.store_compressed(candv.at[pl.ds(cnt, 16)], v, mask=m)
            plsc.store_compressed(candi.at[pl.ds(cnt, 16)], idxv, mask=m)
            pc = plsc.all_reduce_population_count(m)
            cnt = cnt + pc[0]
            R, cnt = lax.cond(
                cnt >= CAND_CAP - 16,
                lambda: (flush(R, cnt), jnp.int32(0)),
                lambda: (R, cnt))
            return (*R, cnt)

        carry = lax.fori_loop(0, nsub, ph_d, (*R, jnp.int32(0)))
        R = flush(carry[:8], carry[8])

        for i in range(4):
            stage_a[tt, pl.ds(i * 16, 16)] = R[2 * i]
            stage_i[tt, pl.ds(i * 16, 16)] = R[2 * i + 1]

        act_vregs = [R[0], R[2], R[4], R[6]]
        wbufs = [wrows0, wrows1]
        sems = [sem0, sem1]

        def batch_idx(b):
            return stage_i.at[tt, pl.ds(b * NROWS, NROWS)]

        def decode_with(acc_ref, osem, esq=esq):
            @pl.when(tt >= 2)
            def _drain():
                pltpu.make_async_copy(acc_ref, out_hbm.at[t - 2], osem).wait()

            cp = pltpu.async_copy(wdec_hbm.at[batch_idx(0)], wbufs[0], sem0)
            for b in range(NBATCH):
                if b + 1 < NBATCH:
                    cp_next = pltpu.async_copy(
                        wdec_hbm.at[batch_idx(b + 1)], wbufs[(b + 1) % 2],
                        sems[(b + 1) % 2])
                cp.wait()
                wbuf = wbufs[b % 2]
                a = act_vregs[b // 2]
                lane0 = (b % 2) * NROWS
                splats = [jnp.full((16,), a[lane0 + r], jnp.float32)
                          for r in range(NROWS)]

                def fma_chunk(c, _, wbuf=wbuf, splats=splats, first=(b == 0)):
                    off = c * 16
                    terms = [splats[r] * wbuf[r, pl.ds(off, 16)]
                             for r in range(NROWS)]
                    if first:
                        terms.append(bdecbuf[pl.ds(off, 16)])
                    while len(terms) > 1:
                        terms = [terms[i] + terms[i + 1]
                                 for i in range(0, len(terms) - 1, 2)] + \
                                ([terms[-1]] if len(terms) % 2 else [])
                    if first:
                        acc_ref[pl.ds(off, 16)] = terms[0]
                    else:
                        acc_ref[pl.ds(off, 16)] += terms[0]
                    return 0
                lax.fori_loop(0, DCH, fma_chunk, 0, unroll=4)
                if b + 1 < NBATCH:
                    cp = cp_next

            cpx.wait()

            def esq_chunk(c, esq):
                off = c * 16
                e = acc_ref[pl.ds(off, 16)] - xbuf[pl.ds(off, 16)]
                return esq + e * e
            esq = lax.fori_loop(0, DCH, esq_chunk, esq, unroll=4)
            pltpu.async_copy(acc_ref, out_hbm.at[t], osem)
            return esq

        return lax.cond(
            par == 0,
            lambda: decode_with(acc0, semo0),
            lambda: decode_with(acc1, semo1))

    esq = lax.fori_loop(0, TPW, token_step, jnp.zeros((16,), jnp.float32))
    pltpu.make_async_copy(acc0, out_hbm.at[t0 + TPW - 2], semo0).wait()
    pltpu.make_async_copy(acc1, out_hbm.at[t0 + TPW - 1], semo1).wait()
    esqbuf[...] = esq
    pltpu.sync_copy(stage_a, acts_hbm.at[pl.ds(t0, TPW)])
    pltpu.sync_copy(stage_i, idx_hbm.at[pl.ds(t0, TPW)])
    pltpu.sync_copy(esqbuf, esq_hbm.at[wid])


def _sc_topk_decode(pre_acts, x, W_dec, b_dec):
    mesh = plsc.VectorSubcoreMesh(
        core_axis_name="c", subcore_axis_name="s",
        num_cores=NC, num_subcores=NS)
    fn = pl.kernel(
        _sc_body,
        out_type=(
            jax.ShapeDtypeStruct((CT, K), jnp.float32),
            jax.ShapeDtypeStruct((CT, K), jnp.int32),
            jax.ShapeDtypeStruct((CT, D), jnp.float32),
            jax.ShapeDtypeStruct((NW, 16), jnp.float32),
        ),
        mesh=mesh,
        compiler_params=pltpu.CompilerParams(needs_layout_passes=False),
        scratch_types=[
            pltpu.VMEM((2, L), jnp.float32),
            pltpu.VMEM((NSUB,), jnp.float32),
            pltpu.VMEM((NSUB + 16,), jnp.int32),
            pltpu.VMEM((CAND_CAP + 16,), jnp.float32),
            pltpu.VMEM((CAND_CAP + 16,), jnp.int32),
            pltpu.VMEM((TPW, K), jnp.float32),
            pltpu.VMEM((TPW, K), jnp.int32),
            pltpu.VMEM((NROWS, D), jnp.float32),
            pltpu.VMEM((NROWS, D), jnp.float32),
            pltpu.VMEM((D,), jnp.float32),
            pltpu.VMEM((D,), jnp.float32),
            pltpu.VMEM((D,), jnp.float32),
            pltpu.VMEM((D,), jnp.float32),
            pltpu.VMEM((16,), jnp.float32),
            pltpu.SemaphoreType.DMA,
            pltpu.SemaphoreType.DMA,
            pltpu.SemaphoreType.DMA,
            pltpu.SemaphoreType.DMA,
            pltpu.SemaphoreType.DMA,
            pltpu.SemaphoreType.DMA,
            pltpu.SemaphoreType.DMA,
        ],
    )
    return fn(pre_acts, x, W_dec, b_dec)


def kernel(x, W_enc, b_enc, W_dec, b_dec):
    sae_in = x - b_dec[None, :]

    enc = pl.pallas_call(
        _enc_body,
        grid=(L // _BL_ENC,),
        in_specs=[
            pl.BlockSpec((CT, D), lambda j: (0, 0)),
            pl.BlockSpec((_BL_ENC, D), lambda j: (j, 0)),
            pl.BlockSpec((1, _BL_ENC), lambda j: (0, j)),
        ],
        out_specs=pl.BlockSpec((CT, _BL_ENC), lambda j: (0, j)),
        out_shape=jax.ShapeDtypeStruct((CT, L), jnp.float32),
    )
    b_enc2 = b_enc.reshape(1, L)

    acts_c, idx_c, out_c, esq_c = [], [], [], []
    for c in range(NCHUNK):
        xc = lax.slice(x, (c * CT, 0), ((c + 1) * CT, D))
        pre_c = enc(lax.slice(sae_in, (c * CT, 0), ((c + 1) * CT, D)),
                    W_enc, b_enc2)
        a, i, o, e = _sc_topk_decode(pre_c, xc, W_dec, b_dec)
        acts_c.append(a)
        idx_c.append(i)
        out_c.append(o)
        esq_c.append(e)

    top_acts = jnp.concatenate(acts_c, axis=0)
    top_indices = jnp.concatenate(idx_c, axis=0)
    sae_out = jnp.concatenate(out_c, axis=0)

    total_variance = pl.pallas_call(
        _xstats_body,
        grid=(4,),
        in_specs=[pl.BlockSpec((T, D // 4), lambda j: (0, j))],
        out_specs=pl.BlockSpec(memory_space=pltpu.SMEM),
        out_shape=jax.ShapeDtypeStruct((1, 1), jnp.float32),
    )(x)[0, 0]

    fvu = sum(jnp.sum(e) for e in esq_c) / total_variance
    auxk_loss = jnp.zeros(())
    multi_topk_fvu = jnp.zeros(())
    return sae_out, top_acts, top_indices, fvu, auxk_loss, multi_topk_fvu


# speedup vs baseline: 1.1405x; 1.0748x over previous
import functools

import jax
import jax.numpy as jnp
from jax import lax
from jax.experimental import pallas as pl
from jax.experimental.pallas import tpu as pltpu
from jax.experimental.pallas import tpu_sc as plsc

T = 2048
D = 2048
L = 32768
K = 64

NC = 2
NS = 16
NW = NC * NS
NCHUNK = 4
CT = T // NCHUNK
TPW = CT // NW
NSUB = L // 16
NGRP = NSUB // 16
CAND_CAP = 1024
NEG = -3.0e38
NROWS = 16
NBATCH = K // NROWS
DCH = D // 16

_BL_ENC = 512
_CTM = 128


def _enc_body(x_ref, w_ref, b_ref, o_ref):
    acc = jax.lax.dot_general(
        x_ref[...], w_ref[...], (((1,), (1,)), ((), ())),
        preferred_element_type=jnp.float32)
    o_ref[...] = jnp.maximum(acc + b_ref[...], 0.0)


def _xstats_body(x_ref, o_ref):
    j = pl.program_id(0)

    @pl.when(j == 0)
    def _init():
        o_ref[0, 0] = 0.0

    xb = x_ref[...]
    s = jnp.sum(xb, axis=0)
    sq = jnp.sum(xb * xb, axis=0)
    o_ref[0, 0] += jnp.sum(sq - s * s * (1.0 / T))


def _merge2(ka, va, kb, vb):
    kbr = lax.rev(kb, (0,))
    vbr = lax.rev(vb, (0,))
    m = ka >= kbr
    khi = jnp.where(m, ka, kbr)
    vhi = jnp.where(m, va, vbr)
    klo = jnp.where(m, kbr, ka)
    vlo = jnp.where(m, vbr, va)
    khi, vhi = plsc.sort_key_val(khi, vhi, descending=True)
    klo, vlo = plsc.sort_key_val(klo, vlo, descending=True)
    return khi, vhi, klo, vlo


def _cascade_insert(R, kv, vv):
    ck, cv = plsc.sort_key_val(kv, vv, descending=True)
    out = []
    for i in range(4):
        khi, vhi, ck, cv = _merge2(R[2 * i], R[2 * i + 1], ck, cv)
        out.append(khi)
        out.append(vhi)
    return tuple(out)


def _maybe_insert(R, kv, vv, gate_splat):
    pc = plsc.all_reduce_population_count(kv >= gate_splat)
    return lax.cond(pc[0] > 0,
                    lambda: _cascade_insert(R, kv, vv),
                    lambda: R)


def _sc_body(pre_hbm, x_hbm, wdec_hbm, bdec_hbm,
             acts_hbm, idx_hbm, out_hbm, esq_hbm,
             databuf, bmbuf, subbuf, candv, candi, stage_a, stage_i,
             wrows0, wrows1, acc0, acc1, xbuf, bdecbuf, esqbuf,
             sem0, sem1, semx, semo0, semo1):
    cid = lax.axis_index("c")
    sid = lax.axis_index("s")
    wid = sid * NC + cid
    t0 = wid * TPW
    iota = lax.iota(jnp.int32, 16)
    neg_splat = jnp.full((16,), NEG, jnp.float32)
    zero_i = jnp.zeros((16,), jnp.int32)

    pltpu.sync_copy(bdec_hbm, bdecbuf)

    def token_step(tt, esq):
        t = t0 + tt
        par = tt % 2
        pltpu.sync_copy(pre_hbm.at[t], databuf)
        cpx = pltpu.async_copy(x_hbm.at[t], xbuf, semx)

        def ph_a(g, _):
            base = g * 256
            m = databuf[pl.ds(base, 16)]
            for i in range(1, 16):
                m = jnp.maximum(m, databuf[pl.ds(base + i * 16, 16)])
            bmbuf[pl.ds(g * 16, 16)] = m
            return 0
        lax.fori_loop(0, NGRP, ph_a, 0, unroll=2)

        bm2 = []
        for i in range(8):
            m = bmbuf[pl.ds(i * 256, 16)]
            for j in range(1, 16):
                m = jnp.maximum(m, bmbuf[pl.ds(i * 256 + j * 16, 16)])
            bm2.append(m)

        R = (neg_splat, zero_i) * 4
        for i in range(8):
            R = _cascade_insert(R, bm2[i], zero_i)
        tau0 = R[6][15]
        tau0_splat = jnp.full((16,), tau0, jnp.float32)

        def ph_c(g, cnt):
            v = bmbuf[pl.ds(g * 16, 16)]
            m = v >= tau0_splat
            plsc.store_compressed(subbuf.at[pl.ds(cnt, 16)], iota + g * 16,
                                  mask=m)
            pc = plsc.all_reduce_population_count(m)
            return cnt + pc[0]
        nsub = lax.fori_loop(0, NGRP, ph_c, jnp.int32(0))

        def flush(R, cnt):
            nv = (cnt + 15) // 16
            def fl_body(q, R):
                base = q * 16
                kv = candv[pl.ds(base, 16)]
                vv = candi[pl.ds(base, 16)]
                valid = (iota + base) < cnt
                kv = jnp.where(valid, kv, neg_splat)
                gate = jnp.full((16,), R[6][15], jnp.float32)
                return _maybe_insert(R, kv, vv, gate)
            return lax.fori_loop(0, nv, fl_body, R)

        R = (neg_splat, zero_i) * 4

        def ph_d(j, carry):
            R = carry[:8]
            cnt = carry[8]
            sub = subbuf[pl.ds(j, 16)][0]
            base = (sub >> 4) * 256 + (sub & 15)
            idxv = base + iota * 16
            v = plsc.load_gather(databuf, [idxv])
            m = v >= tau0_splat
            plsc.store_compressed(candv.at[pl.ds(cnt, 16)], v, mask=m)
            plsc.store_compressed(candi.at[pl.ds(cnt, 16)], idxv, mask=m)
            pc = plsc.all_reduce_population_count(m)
            cnt = cnt + pc[0]
            R, cnt = lax.cond(
                cnt >= CAND_CAP - 16,
                lambda: (flush(R, cnt), jnp.int32(0)),
                lambda: (R, cnt))
            return (*R, cnt)

        carry = lax.fori_loop(0, nsub, ph_d, (*R, jnp.int32(0)))
        R = flush(carry[:8], carry[8])

        for i in range(4):
            stage_a[tt, pl.ds(i * 16, 16)] = R[2 * i]
            stage_i[tt, pl.ds(i * 16, 16)] = R[2 * i + 1]

        idx_vregs = [R[1], R[3], R[5], R[7]]
        act_vregs = [R[0], R[2], R[4], R[6]]
        wbufs = [wrows0, wrows1]
        sems = [sem0, sem1]

        def batch_idx(b):
            return idx_vregs[b]

        def decode_with(acc_ref, osem, esq=esq):
            @pl.when(tt >= 2)
            def _drain():
                pltpu.make_async_copy(acc_ref, out_hbm.at[t - 2], osem).wait()

            cp = pltpu.async_copy(wdec_hbm.at[batch_idx(0)], wbufs[0], sem0)
            for b in range(NBATCH):
                if b + 1 < NBATCH:
                    cp_next = pltpu.async_copy(
                        wdec_hbm.at[batch_idx(b + 1)], wbufs[(b + 1) % 2],
                        sems[(b + 1) % 2])
                cp.wait()
                wbuf = wbufs[b % 2]
                a = act_vregs[b]
                splats = [jnp.full((16,), a[r], jnp.float32)
                          for r in range(NROWS)]

                def fma_chunk(c, _, wbuf=wbuf, splats=splats, first=(b == 0)):
                    off = c * 16
                    terms = [splats[r] * wbuf[r, pl.ds(off, 16)]
                             for r in range(NROWS)]
                    if first:
                        terms.append(bdecbuf[pl.ds(off, 16)])
                    while len(terms) > 1:
                        terms = [terms[i] + terms[i + 1]
                                 for i in range(0, len(terms) - 1, 2)] + \
                                ([terms[-1]] if len(terms) % 2 else [])
                    if first:
                        acc_ref[pl.ds(off, 16)] = terms[0]
                    else:
                        acc_ref[pl.ds(off, 16)] += terms[0]
                    return 0
                lax.fori_loop(0, DCH, fma_chunk, 0, unroll=4)
                if b + 1 < NBATCH:
                    cp = cp_next

            cpx.wait()

            def esq_chunk(c, esq):
                off = c * 16
                e = acc_ref[pl.ds(off, 16)] - xbuf[pl.ds(off, 16)]
                return esq + e * e
            esq = lax.fori_loop(0, DCH, esq_chunk, esq, unroll=4)
            pltpu.async_copy(acc_ref, out_hbm.at[t], osem)
            return esq

        return lax.cond(
            par == 0,
            lambda: decode_with(acc0, semo0),
            lambda: decode_with(acc1, semo1))

    esq = lax.fori_loop(0, TPW, token_step, jnp.zeros((16,), jnp.float32))
    pltpu.make_async_copy(acc0, out_hbm.at[t0 + TPW - 2], semo0).wait()
    pltpu.make_async_copy(acc1, out_hbm.at[t0 + TPW - 1], semo1).wait()
    esqbuf[...] = esq
    pltpu.sync_copy(stage_a, acts_hbm.at[pl.ds(t0, TPW)])
    pltpu.sync_copy(stage_i, idx_hbm.at[pl.ds(t0, TPW)])
    pltpu.sync_copy(esqbuf, esq_hbm.at[wid])


def _sc_topk_decode(pre_acts, x, W_dec, b_dec):
    mesh = plsc.VectorSubcoreMesh(
        core_axis_name="c", subcore_axis_name="s",
        num_cores=NC, num_subcores=NS)
    fn = pl.kernel(
        _sc_body,
        out_type=(
            jax.ShapeDtypeStruct((CT, K), jnp.float32),
            jax.ShapeDtypeStruct((CT, K), jnp.int32),
            jax.ShapeDtypeStruct((CT, D), jnp.float32),
            jax.ShapeDtypeStruct((NW, 16), jnp.float32),
        ),
        mesh=mesh,
        compiler_params=pltpu.CompilerParams(needs_layout_passes=False),
        scratch_types=[
            pltpu.VMEM((L,), jnp.float32),
            pltpu.VMEM((NSUB,), jnp.float32),
            pltpu.VMEM((NSUB + 16,), jnp.int32),
            pltpu.VMEM((CAND_CAP + 16,), jnp.float32),
            pltpu.VMEM((CAND_CAP + 16,), jnp.int32),
            pltpu.VMEM((TPW, K), jnp.float32),
            pltpu.VMEM((TPW, K), jnp.int32),
            pltpu.VMEM((NROWS, D), jnp.float32),
            pltpu.VMEM((NROWS, D), jnp.float32),
            pltpu.VMEM((D,), jnp.float32),
            pltpu.VMEM((D,), jnp.float32),
            pltpu.VMEM((D,), jnp.float32),
            pltpu.VMEM((D,), jnp.float32),
            pltpu.VMEM((16,), jnp.float32),
            pltpu.SemaphoreType.DMA,
            pltpu.SemaphoreType.DMA,
            pltpu.SemaphoreType.DMA,
            pltpu.SemaphoreType.DMA,
            pltpu.SemaphoreType.DMA,
        ],
    )
    return fn(pre_acts, x, W_dec, b_dec)


def kernel(x, W_enc, b_enc, W_dec, b_dec):
    sae_in = x - b_dec[None, :]

    enc = pl.pallas_call(
        _enc_body,
        grid=(L // _BL_ENC,),
        in_specs=[
            pl.BlockSpec((CT, D), lambda j: (0, 0)),
            pl.BlockSpec((_BL_ENC, D), lambda j: (j, 0)),
            pl.BlockSpec((1, _BL_ENC), lambda j: (0, j)),
        ],
        out_specs=pl.BlockSpec((CT, _BL_ENC), lambda j: (0, j)),
        out_shape=jax.ShapeDtypeStruct((CT, L), jnp.float32),
    )
    b_enc2 = b_enc.reshape(1, L)

    acts_c, idx_c, out_c, esq_c = [], [], [], []
    for c in range(NCHUNK):
        xc = lax.slice(x, (c * CT, 0), ((c + 1) * CT, D))
        pre_c = enc(lax.slice(sae_in, (c * CT, 0), ((c + 1) * CT, D)),
                    W_enc, b_enc2)
        a, i, o, e = _sc_topk_decode(pre_c, xc, W_dec, b_dec)
        acts_c.append(a)
        idx_c.append(i)
        out_c.append(o)
        esq_c.append(e)

    top_acts = jnp.concatenate(acts_c, axis=0)
    top_indices = jnp.concatenate(idx_c, axis=0)
    sae_out = jnp.concatenate(out_c, axis=0)

    total_variance = pl.pallas_call(
        _xstats_body,
        grid=(4,),
        in_specs=[pl.BlockSpec((T, D // 4), lambda j: (0, j))],
        out_specs=pl.BlockSpec(memory_space=pltpu.SMEM),
        out_shape=jax.ShapeDtypeStruct((1, 1), jnp.float32),
    )(x)[0, 0]

    fvu = sum(jnp.sum(e) for e in esq_c) / total_variance
    auxk_loss = jnp.zeros(())
    multi_topk_fvu = jnp.zeros(())
    return sae_out, top_acts, top_indices, fvu, auxk_loss, multi_topk_fvu
